# baseline (speedup 1.0000x reference)
import functools

import jax
import jax.numpy as jnp
from jax import lax
from jax.experimental import pallas as pl
from jax.experimental.pallas import tpu as pltpu
from jax.experimental.pallas import tpu_sc as plsc

VOCAB = 1000000
EMBED = 64
B, L = 4096, 200
NC = 2
NS = 16
NW = NC * NS
NBLK_A = VOCAB // 128
JB = B // 128

_mesh = plsc.VectorSubcoreMesh(core_axis_name="c", subcore_axis_name="s")
_params = pltpu.CompilerParams(
    use_tc_tiling_on_sc=True, needs_layout_passes=False)

_IOTA16 = [
    tuple(range(k * 16, k * 16 + 16)) for k in range(8)
]


def _iota(k):
    return jnp.arange(k * 16, k * 16 + 16, dtype=jnp.int32)


def _scat_row(k):
    return jax.lax.shift_right_logical(_iota(k), 1)


def _scat_col(k):
    return jax.lax.shift_left(jax.lax.bitwise_and(_iota(k), 1), 6)


@functools.partial(
    pl.kernel,
    mesh=_mesh,
    out_type=jax.ShapeDtypeStruct((VOCAB // 2, 128), jnp.float32),
    scratch_types=[
        pltpu.VMEM((64, 136), jnp.float32),
        pltpu.VMEM((64, 136), jnp.float32),
        pltpu.VMEM((64, 128), jnp.float32),
        pltpu.SemaphoreType.DMA,
        pltpu.SemaphoreType.DMA,
    ],
    compiler_params=_params,
)
def _table_transpose(tt_hbm, tail_hbm, tl_hbm, src0, src1, tbuf, rsem0, rsem1):
    wid = lax.axis_index("s") * NC + lax.axis_index("c")
    srcs = (src0, src1)
    rsems = (rsem0, rsem1)
    nblk = jnp.where(wid < 4, 245, 244).astype(jnp.int32)

    def _start_read(k, buf_i):
        j = wid + k * NW
        off = pl.multiple_of(j * 128, 128)
        return pltpu.async_copy(
            tt_hbm.at[:, pl.ds(off, 128)],
            srcs[buf_i].at[:, pl.ds(0, 128)], rsems[buf_i])

    _start_read(0, 0)

    def body(k, carry):
        for t in range(2):
            kk = k * 2 + t

            @pl.when(kk < nblk)
            def _():
                @pl.when(kk + 1 < nblk)
                def _():
                    _start_read(kk + 1, 1 - t)

                pltpu.make_async_copy(
                    tt_hbm.at[:, pl.ds(0, 128)],
                    srcs[t].at[:, pl.ds(0, 128)], rsems[t]).wait()
                src = srcs[t]

                @plsc.parallel_loop(0, 64, unroll=1)
                def trow(e):
                    for k in range(8):
                        v = src[e, pl.ds(k * 16, 16)]
                        plsc.store_scatter(tbuf, [_scat_row(k), _scat_col(k) + e], v)
                j = wid + kk * NW
                woff = pl.multiple_of(j * 64, 64)
                pltpu.sync_copy(tbuf, tl_hbm.at[pl.ds(woff, 64)])

        return carry

    lax.fori_loop(0, 123, body, 0)

    @pl.when(wid == 4)
    def _():
        pltpu.sync_copy(tail_hbm, tbuf.at[pl.ds(0, 32)])
        pltpu.sync_copy(tbuf.at[pl.ds(0, 32)], tl_hbm.at[pl.ds(499968, 32)])


@functools.partial(
    pl.kernel,
    mesh=_mesh,
    out_type=jax.ShapeDtypeStruct((L, 8, JB, 8, 128), jnp.float32),
    scratch_types=[
        pltpu.VMEM((L, 128), jnp.int32),
        pltpu.VMEM((128,), jnp.int32),
        pltpu.VMEM((128,), jnp.int32),
        pltpu.VMEM((128, 136), jnp.float32),
        pltpu.VMEM((128, 136), jnp.float32),
        pltpu.VMEM((8, 8, 128), jnp.float32),
        pltpu.SemaphoreType.DMA,
        pltpu.SemaphoreType.DMA,
        pltpu.SemaphoreType.DMA,
    ],
    compiler_params=_params,
)
def _gather_tiles(idx_hbm, tl_hbm, out_hbm, idx_v, h0, h1, p0, p1, tbuf,
                  gsem0, gsem1, wsem):
    wid = lax.axis_index("s") * NC + lax.axis_index("c")
    hs = (h0, h1)
    ps = (p0, p1)
    gsems = (gsem0, gsem1)
    pltpu.sync_copy(idx_hbm.at[wid], idx_v)

    def _start_gather(l, buf_i):
        for cc in range(8):
            raw = idx_v[l, pl.ds(cc * 16, 16)]
            hs[buf_i][pl.ds(cc * 16, 16)] = jax.lax.shift_right_logical(raw, 1)
        pltpu.async_copy(
            tl_hbm.at[hs[buf_i]], ps[buf_i].at[:, pl.ds(0, 128)],
            gsems[buf_i])

    _start_gather(0, 0)

    def body(l2, carry):
        for t in range(2):
            l = l2 * 2 + t

            @pl.when(l + 1 < L)
            def _():
                _start_gather(l + 1, 1 - t)

            pltpu.make_async_copy(
                tl_hbm.at[hs[t]], ps[t].at[:, pl.ds(0, 128)],
                gsems[t]).wait()
            pairs = ps[t]

            @pl.when(l >= 1)
            def _():
                pltpu.make_async_copy(
                    tbuf, out_hbm.at[0, :, 0], wsem).wait()

            @plsc.parallel_loop(0, 8, unroll=1)
            def tcc(cc):
                raw = idx_v[l, pl.ds(cc * 16, 16)]
                hoff = jax.lax.shift_left(
                    jax.lax.bitwise_and(raw, 1), 6)
                rowv = jnp.arange(16, dtype=jnp.int32) + cc * 16
                for eh in range(8):
                    for r in range(8):
                        col = hoff + (8 * eh + r)
                        v = plsc.load_gather(pairs, [rowv, col])
                        tbuf[eh, r, pl.ds(cc * 16, 16)] = v
            pltpu.async_copy(tbuf, out_hbm.at[l, :, wid], wsem)

        return carry

    lax.fori_loop(0, L // 2, body, 0)
    pltpu.make_async_copy(tbuf, out_hbm.at[0, :, 0], wsem).wait()


def kernel(x, table):
    table_t = table.T
    tail_l = table[VOCAB - 64:].reshape(32, 128)
    table_l = _table_transpose(table_t, tail_l)
    idx3 = (
        x.astype(jnp.int32).reshape(JB, 128, L).transpose(0, 2, 1)
    )
    out5 = _gather_tiles(idx3, table_l)
    return out5.transpose(2, 4, 0, 1, 3).reshape(B, L, EMBED)


# speedup vs baseline: 1.4259x; 1.4259x over previous
import functools

import jax
import jax.numpy as jnp
from jax import lax
from jax.experimental import pallas as pl
from jax.experimental.pallas import tpu as pltpu
from jax.experimental.pallas import tpu_sc as plsc

VOCAB = 1000000
EMBED = 64
B, L = 4096, 200
N = B * L
NC = 2
NS = 16
NW = NC * NS
BPW = N // NW
CH = 128
NCH = BPW // CH

_mesh = plsc.VectorSubcoreMesh(core_axis_name="c", subcore_axis_name="s")
_params = pltpu.CompilerParams(
    use_tc_tiling_on_sc=True, needs_layout_passes=False)


@functools.partial(
    pl.kernel,
    mesh=_mesh,
    out_type=jax.ShapeDtypeStruct((N, EMBED), jnp.float32),
    scratch_types=[
        pltpu.VMEM((NCH, CH), jnp.int32),
        pltpu.VMEM((CH,), jnp.int32),
        pltpu.VMEM((CH,), jnp.int32),
        pltpu.VMEM((CH, 128), jnp.float32),
        pltpu.VMEM((CH, 128), jnp.float32),
        pltpu.VMEM((CH, EMBED), jnp.float32),
        pltpu.SemaphoreType.DMA,
        pltpu.SemaphoreType.DMA,
        pltpu.SemaphoreType.DMA,
    ],
    compiler_params=_params,
)
def _gather_rows(idx_hbm, tl_hbm, out_hbm, idx_v, h0, h1, p0, p1, rows_v,
                 gsem0, gsem1, wsem):
    wid = lax.axis_index("s") * NC + lax.axis_index("c")
    base = wid * BPW
    hs = (h0, h1)
    ps = (p0, p1)
    gsems = (gsem0, gsem1)
    pltpu.sync_copy(idx_hbm.at[wid], idx_v)

    def _start_gather(j, buf_i):
        for cc in range(8):
            raw = idx_v[j, pl.ds(cc * 16, 16)]
            hs[buf_i][pl.ds(cc * 16, 16)] = jax.lax.shift_right_logical(raw, 1)
        pltpu.async_copy(tl_hbm.at[hs[buf_i]], ps[buf_i], gsems[buf_i])

    _start_gather(0, 0)

    def body(j2, carry):
        for t in range(2):
            j = j2 * 2 + t

            @pl.when(j + 1 < NCH)
            def _():
                _start_gather(j + 1, 1 - t)

            pltpu.make_async_copy(tl_hbm.at[hs[t]], ps[t], gsems[t]).wait()
            pairs = ps[t]

            @pl.when(j >= 1)
            def _():
                pltpu.make_async_copy(
                    rows_v, out_hbm.at[pl.ds(0, CH)], wsem).wait()

            def sel(g, carry2):
                raw16 = idx_v[j, pl.ds(g * 16, 16)]
                h64v = jax.lax.shift_left(jax.lax.bitwise_and(raw16, 1), 6)
                for i in range(16):
                    c = g * 16 + i
                    h64 = h64v[i]
                    for m in range(4):
                        rows_v[c, pl.ds(m * 16, 16)] = (
                            pairs[c, pl.ds(h64 + m * 16, 16)])
                return carry2

            lax.fori_loop(0, CH // 16, sel, 0)
            pltpu.async_copy(
                rows_v, out_hbm.at[pl.ds(base + j * CH, CH)], wsem)

        return carry

    lax.fori_loop(0, NCH // 2, body, 0)
    pltpu.make_async_copy(rows_v, out_hbm.at[pl.ds(0, CH)], wsem).wait()


def kernel(x, table):
    tl = table.reshape(VOCAB // 2, 128)
    idx3 = x.astype(jnp.int32).reshape(NW, NCH, CH)
    out = _gather_rows(idx3, tl)
    return out.reshape(B, L, EMBED)


# speedup vs baseline: 1.7038x; 1.1949x over previous
import functools

import jax
import jax.numpy as jnp
from jax import lax
from jax.experimental import pallas as pl
from jax.experimental.pallas import tpu as pltpu
from jax.experimental.pallas import tpu_sc as plsc

VOCAB = 1000000
EMBED = 64
B, L = 4096, 200
N = B * L
NC = 2
NS = 16
NW = NC * NS
BPW = N // NW
CH = 128
NCH = BPW // CH

_mesh = plsc.VectorSubcoreMesh(core_axis_name="c", subcore_axis_name="s")
_params = pltpu.CompilerParams(
    use_tc_tiling_on_sc=True, needs_layout_passes=False)


@functools.partial(
    pl.kernel,
    mesh=_mesh,
    out_type=jax.ShapeDtypeStruct((N, EMBED), jnp.float32),
    scratch_types=[
        pltpu.VMEM((NCH, CH), jnp.int32),
        pltpu.VMEM((CH,), jnp.int32),
        pltpu.VMEM((CH,), jnp.int32),
        pltpu.VMEM((CH, 1, 2 * EMBED), jnp.float32),
        pltpu.VMEM((CH, 1, 2 * EMBED), jnp.float32),
        pltpu.VMEM((CH, EMBED), jnp.float32),
        pltpu.SemaphoreType.DMA,
        pltpu.SemaphoreType.DMA,
        pltpu.SemaphoreType.DMA,
    ],
    compiler_params=_params,
)
def _gather_rows(idx_hbm, t3_hbm, out_hbm, idx_v, h0, h1, p0, p1, rows_v,
                 gsem0, gsem1, wsem):
    wid = lax.axis_index("s") * NC + lax.axis_index("c")
    base = wid * BPW
    hs = (h0, h1)
    ps = (p0, p1)
    gsems = (gsem0, gsem1)
    pltpu.sync_copy(idx_hbm.at[wid], idx_v)

    def _start_gather(j, buf_i):
        for cc in range(8):
            raw = idx_v[j, pl.ds(cc * 16, 16)]
            hs[buf_i][pl.ds(cc * 16, 16)] = jax.lax.shift_right_logical(raw, 1)
        pltpu.async_copy(t3_hbm.at[hs[buf_i]], ps[buf_i], gsems[buf_i])

    _start_gather(0, 0)

    def body(j2, carry):
        for t in range(2):
            j = j2 * 2 + t

            @pl.when(j + 1 < NCH)
            def _():
                _start_gather(j + 1, 1 - t)

            pltpu.make_async_copy(t3_hbm.at[hs[t]], ps[t], gsems[t]).wait()
            slabs = ps[t]

            @pl.when(j >= 1)
            def _():
                pltpu.make_async_copy(
                    rows_v, out_hbm.at[pl.ds(0, CH)], wsem).wait()

            @plsc.parallel_loop(0, CH // 16, unroll=2)
            def sel(g):
                hv = jax.lax.shift_left(
                    jax.lax.bitwise_and(idx_v[j, pl.ds(g * 16, 16)], 1), 6)
                for i in range(16):
                    c = g * 16 + i
                    h64 = hv[i]
                    for m in range(4):
                        rows_v[c, pl.ds(m * 16, 16)] = (
                            slabs[c, 0, pl.ds(h64 + m * 16, 16)])

            pltpu.async_copy(
                rows_v, out_hbm.at[pl.ds(base + j * CH, CH)], wsem)

        return carry

    lax.fori_loop(0, NCH // 2, body, 0)
    pltpu.make_async_copy(rows_v, out_hbm.at[pl.ds(0, CH)], wsem).wait()


def kernel(x, table):
    t3 = table.reshape(VOCAB // 2, 1, 2 * EMBED)
    idx3 = x.astype(jnp.int32).reshape(NW, NCH, CH)
    out = _gather_rows(idx3, t3)
    return out.reshape(B, L, EMBED)
